# baseline (speedup 1.0000x reference)
import functools

import jax
import jax.numpy as jnp
from jax import lax
from jax.experimental import pallas as pl
from jax.experimental.pallas import tpu as pltpu, tpu_sc as plsc

N = 10000
E = 640000
DIN, DH, DOUT = 128, 128, 64

NC, NS = 2, 16
NW = NC * NS
C = 128
CHUNKS = 158
EW = CHUNKS * C
EPT = NW * EW
NP = 10240
RPT = NP // NS

_mesh = plsc.VectorSubcoreMesh(core_axis_name="c", subcore_axis_name="s")


@functools.partial(
    pl.kernel,
    out_type=jax.ShapeDtypeStruct((NC, NP), jnp.float32),
    mesh=_mesh,
    scratch_types=[
        pltpu.VMEM((CHUNKS, C), jnp.int32),
        pltpu.VMEM((C,), jnp.float32),
        pltpu.VMEM((C,), jnp.float32),
        pltpu.VMEM_SHARED((NP,), jnp.float32),
    ],
)
def _sc_deg(dst_hbm, out_hbm, idx_v, ones_v, zero_v, acc):
    c = lax.axis_index("c")
    s = lax.axis_index("s")
    wid = s * NC + c
    pltpu.sync_copy(dst_hbm.at[wid], idx_v)
    for k in range(C // 16):
        ones_v[pl.ds(k * 16, 16)] = jnp.ones((16,), jnp.float32)
        zero_v[pl.ds(k * 16, 16)] = jnp.zeros((16,), jnp.float32)
    for k in range(RPT // C):
        pltpu.sync_copy(zero_v, acc.at[pl.ds(s * RPT + k * C, C)])
    plsc.subcore_barrier()

    def body(j, carry):
        pltpu.sync_copy(ones_v, acc.at[idx_v.at[j]], add=True)
        return carry

    lax.fori_loop(0, CHUNKS, body, 0)
    plsc.subcore_barrier()
    pltpu.sync_copy(acc.at[pl.ds(s * RPT, RPT)],
                    out_hbm.at[c, pl.ds(s * RPT, RPT)])


def _make_sc_agg(d):
    @functools.partial(
        pl.kernel,
        out_type=jax.ShapeDtypeStruct((NC, NP, d), jnp.float32),
        mesh=_mesh,
        scratch_types=[
            pltpu.VMEM((CHUNKS + 2, C), jnp.int32),
            pltpu.VMEM((CHUNKS, C), jnp.int32),
            [pltpu.VMEM((C, d), jnp.float32)] * 4,
            [pltpu.SemaphoreType.DMA] * 4,
            [pltpu.SemaphoreType.DMA] * 4,
            pltpu.VMEM_SHARED((NP, d), jnp.float32),
        ],
        compiler_params=pltpu.CompilerParams(use_tc_tiling_on_sc=False),
    )
    def _sc_agg(y_hbm, src_hbm, dst_hbm, zero_hbm, out_hbm,
                src_v, dst_v, bufs, gsem, ssem, acc):
        c = lax.axis_index("c")
        s = lax.axis_index("s")
        wid = s * NC + c
        pltpu.sync_copy(src_hbm.at[wid], src_v.at[pl.ds(0, CHUNKS)])
        pltpu.sync_copy(dst_hbm.at[wid], dst_v)
        for k in range(C // 16):
            src_v[CHUNKS, pl.ds(k * 16, 16)] = jnp.zeros((16,), jnp.int32)
            src_v[CHUNKS + 1, pl.ds(k * 16, 16)] = jnp.zeros((16,), jnp.int32)
        pltpu.sync_copy(zero_hbm.at[pl.ds(s * RPT, RPT)],
                        acc.at[pl.ds(s * RPT, RPT)])
        plsc.subcore_barrier()

        def gather(j, b):
            pltpu.async_copy(y_hbm.at[src_v.at[j]], bufs[b], gsem[b])

        def gather_wait(j, b):
            pltpu.make_async_copy(y_hbm.at[src_v.at[j]], bufs[b],
                                  gsem[b]).wait()

        def scatter(j, b):
            pltpu.async_copy(bufs[b], acc.at[dst_v.at[j]], ssem[b], add=True)

        def scatter_wait(j, b):
            pltpu.make_async_copy(bufs[b], acc.at[dst_v.at[j]],
                                  ssem[b]).wait()

        gather(0, 0)
        gather(1, 1)
        gather_wait(0, 0)
        scatter(0, 0)
        gather(2, 2)
        gather_wait(1, 1)
        scatter(1, 1)
        gather(3, 3)

        def body(i, carry):
            j0 = 2 + i * 4
            for t in range(4):
                j = j0 + t
                b = (2 + t) % 4
                gather_wait(j, b)
                scatter(j, b)
                scatter_wait(j - 2, (b + 2) % 4)
                gather(j + 2, (b + 2) % 4)
            return carry

        lax.fori_loop(0, (CHUNKS - 2) // 4, body, 0)
        scatter_wait(CHUNKS - 2, (CHUNKS - 2) % 4)
        scatter_wait(CHUNKS - 1, (CHUNKS - 1) % 4)
        gather_wait(CHUNKS, CHUNKS % 4)
        gather_wait(CHUNKS + 1, (CHUNKS + 1) % 4)
        plsc.subcore_barrier()
        pltpu.sync_copy(acc.at[pl.ds(s * RPT, RPT)],
                        out_hbm.at[c, pl.ds(s * RPT, RPT)])

    return _sc_agg


_sc_agg_dout = _make_sc_agg(DOUT)
_sc_agg_d32 = _make_sc_agg(32)


_BR = 1024


_DHH = DH // 2


def _tc_a_body(x_ref, w_ref, d_ref, oa_ref, ob_ref):
    deg = d_ref[:, 0:1] + d_ref[:, 1:2] + 1.0
    dinv = lax.rsqrt(deg)
    xw = jnp.dot(x_ref[...], w_ref[...], preferred_element_type=jnp.float32)
    y = xw * dinv
    oa_ref[...] = y[:, :_DHH]
    ob_ref[...] = y[:, _DHH:]


def _tc_a(x_pad, w1, degt):
    return pl.pallas_call(
        _tc_a_body,
        grid=(NP // _BR,),
        in_specs=[
            pl.BlockSpec((_BR, DIN), lambda i: (i, 0)),
            pl.BlockSpec((DIN, DH), lambda i: (0, 0)),
            pl.BlockSpec((_BR, NC), lambda i: (i, 0)),
        ],
        out_specs=[
            pl.BlockSpec((_BR, _DHH), lambda i: (i, 0)),
            pl.BlockSpec((_BR, _DHH), lambda i: (i, 0)),
        ],
        out_shape=[
            jax.ShapeDtypeStruct((NP, _DHH), jnp.float32),
            jax.ShapeDtypeStruct((NP, _DHH), jnp.float32),
        ],
    )(x_pad, w1, degt)


def _tc_b_body(pa_ref, pb_ref, ya_ref, yb_ref, d_ref, b_ref, w_ref, o_ref):
    deg = d_ref[:, 0:1] + d_ref[:, 1:2] + 1.0
    dinv = lax.rsqrt(deg)
    agga = pa_ref[0] + pa_ref[1] + ya_ref[...]
    aggb = pb_ref[0] + pb_ref[1] + yb_ref[...]
    ha = jnp.maximum(agga * dinv + b_ref[:, :_DHH], 0.0)
    hb = jnp.maximum(aggb * dinv + b_ref[:, _DHH:], 0.0)
    hw = (jnp.dot(ha, w_ref[:_DHH, :], preferred_element_type=jnp.float32)
          + jnp.dot(hb, w_ref[_DHH:, :], preferred_element_type=jnp.float32))
    o_ref[...] = hw * dinv


def _tc_b(p1a, p1b, y1a, y1b, degt, b1, w2):
    return pl.pallas_call(
        _tc_b_body,
        grid=(NP // _BR,),
        in_specs=[
            pl.BlockSpec((NC, _BR, _DHH), lambda i: (0, i, 0)),
            pl.BlockSpec((NC, _BR, _DHH), lambda i: (0, i, 0)),
            pl.BlockSpec((_BR, _DHH), lambda i: (i, 0)),
            pl.BlockSpec((_BR, _DHH), lambda i: (i, 0)),
            pl.BlockSpec((_BR, NC), lambda i: (i, 0)),
            pl.BlockSpec((1, DH), lambda i: (0, 0)),
            pl.BlockSpec((DH, DOUT), lambda i: (0, 0)),
        ],
        out_specs=pl.BlockSpec((_BR, DOUT), lambda i: (i, 0)),
        out_shape=jax.ShapeDtypeStruct((NP, DOUT), jnp.float32),
    )(p1a, p1b, y1a, y1b, degt, b1, w2)


_BRC = 1000


def _tc_c_body(p_ref, y_ref, d_ref, b_ref, o_ref):
    deg = d_ref[:, 0:1] + d_ref[:, 1:2] + 1.0
    dinv = lax.rsqrt(deg)
    agg = p_ref[0] + p_ref[1] + y_ref[...]
    o_ref[...] = agg * dinv + b_ref[...]


def _tc_c(p2, y2, degt, b2):
    return pl.pallas_call(
        _tc_c_body,
        grid=(N // _BRC,),
        in_specs=[
            pl.BlockSpec((NC, _BRC, DOUT), lambda i: (0, i, 0)),
            pl.BlockSpec((_BRC, DOUT), lambda i: (i, 0)),
            pl.BlockSpec((_BRC, NC), lambda i: (i, 0)),
            pl.BlockSpec((1, DOUT), lambda i: (0, 0)),
        ],
        out_specs=pl.BlockSpec((_BRC, DOUT), lambda i: (i, 0)),
        out_shape=jax.ShapeDtypeStruct((N, DOUT), jnp.float32),
    )(p2, y2, degt, b2)


def kernel(x, edge_index, W1, b1, W2, b2):
    src = edge_index[0]
    dst = edge_index[1]
    fill = jnp.full((EPT - E,), N, dtype=jnp.int32)
    src_p = jnp.concatenate([src, fill]).reshape(NW, CHUNKS, C)
    dst_p = jnp.concatenate([dst, fill]).reshape(NW, CHUNKS, C)
    x_pad = jnp.pad(x, ((0, NP - N), (0, 0)))

    degp = _sc_deg(dst_p)
    degt = degp.T
    y1a, y1b = _tc_a(x_pad, W1, degt)
    z = jnp.zeros((NP, DOUT), jnp.float32)
    p1a = _sc_agg_dout(y1a, src_p, dst_p, z)
    p1b = _sc_agg_dout(y1b, src_p, dst_p, z)
    y2 = _tc_b(p1a, p1b, y1a, y1b, degt, b1.reshape(1, DH), W2)
    z32 = jnp.zeros((NP, 32), jnp.float32)
    p2a = _sc_agg_d32(y2[:, :32], src_p, dst_p, z32)
    p2b = _sc_agg_d32(y2[:, 32:], src_p, dst_p, z32)
    p2 = jnp.concatenate([p2a, p2b], axis=2)
    out = _tc_c(p2, y2, degt, b2.reshape(1, DOUT))
    return out


# speedup vs baseline: 1.1248x; 1.1248x over previous
import functools

import jax
import jax.numpy as jnp
from jax import lax
from jax.experimental import pallas as pl
from jax.experimental.pallas import tpu as pltpu, tpu_sc as plsc

N = 10000
E = 640000
DIN, DH, DOUT = 128, 128, 64

NC, NS = 2, 16
NW = NC * NS
C = 128
CHUNKS = 158
EW = CHUNKS * C
EPT = NW * EW
NP = 10240
RPT = NP // NS

_mesh = plsc.VectorSubcoreMesh(core_axis_name="c", subcore_axis_name="s")


@functools.partial(
    pl.kernel,
    out_type=jax.ShapeDtypeStruct((NC, NP), jnp.float32),
    mesh=_mesh,
    scratch_types=[
        pltpu.VMEM((CHUNKS, C), jnp.int32),
        pltpu.VMEM((C,), jnp.float32),
        pltpu.VMEM((C,), jnp.float32),
        pltpu.VMEM_SHARED((NP,), jnp.float32),
    ],
)
def _sc_deg(dst_hbm, out_hbm, idx_v, ones_v, zero_v, acc):
    c = lax.axis_index("c")
    s = lax.axis_index("s")
    wid = s * NC + c
    pltpu.sync_copy(dst_hbm.at[wid], idx_v)
    for k in range(C // 16):
        ones_v[pl.ds(k * 16, 16)] = jnp.ones((16,), jnp.float32)
        zero_v[pl.ds(k * 16, 16)] = jnp.zeros((16,), jnp.float32)
    for k in range(RPT // C):
        pltpu.sync_copy(zero_v, acc.at[pl.ds(s * RPT + k * C, C)])
    plsc.subcore_barrier()

    def body(j, carry):
        pltpu.sync_copy(ones_v, acc.at[idx_v.at[j]], add=True)
        return carry

    lax.fori_loop(0, CHUNKS, body, 0)
    plsc.subcore_barrier()
    pltpu.sync_copy(acc.at[pl.ds(s * RPT, RPT)],
                    out_hbm.at[c, pl.ds(s * RPT, RPT)])


def _make_sc_agg(d, npass):
    @functools.partial(
        pl.kernel,
        out_type=[jax.ShapeDtypeStruct((NC, NP, d), jnp.float32)] * npass,
        mesh=_mesh,
        scratch_types=[
            pltpu.VMEM((CHUNKS + 2, C), jnp.int32),
            pltpu.VMEM((CHUNKS, C), jnp.int32),
            [pltpu.VMEM((C, d), jnp.float32)] * 4,
            pltpu.VMEM((C, d), jnp.float32),
            [pltpu.SemaphoreType.DMA] * 4,
            [pltpu.SemaphoreType.DMA] * 4,
            pltpu.VMEM_SHARED((NP, d), jnp.float32),
        ],
        compiler_params=pltpu.CompilerParams(use_tc_tiling_on_sc=False),
    )
    def _sc_agg(*refs):
        ys = refs[:npass]
        src_hbm, dst_hbm = refs[npass], refs[npass + 1]
        outs = refs[npass + 2:2 * npass + 2]
        src_v, dst_v, bufs, zbuf, gsem, ssem, acc = refs[2 * npass + 2:]
        c = lax.axis_index("c")
        s = lax.axis_index("s")
        wid = s * NC + c
        pltpu.sync_copy(src_hbm.at[wid], src_v.at[pl.ds(0, CHUNKS)])
        pltpu.sync_copy(dst_hbm.at[wid], dst_v)
        for k in range(C // 16):
            src_v[CHUNKS, pl.ds(k * 16, 16)] = jnp.zeros((16,), jnp.int32)
            src_v[CHUNKS + 1, pl.ds(k * 16, 16)] = jnp.zeros((16,), jnp.int32)

        def zrow(r, carry):
            for k in range(d // 16):
                zbuf[r, pl.ds(k * 16, 16)] = jnp.zeros((16,), jnp.float32)
            return carry

        lax.fori_loop(0, C, zrow, 0)

        def zero_acc():
            for k in range(RPT // C):
                pltpu.sync_copy(zbuf, acc.at[pl.ds(s * RPT + k * C, C)])

        def ring(y_hbm):
            def gather(j, b):
                pltpu.async_copy(y_hbm.at[src_v.at[j]], bufs[b], gsem[b])

            def gather_wait(j, b):
                pltpu.make_async_copy(y_hbm.at[src_v.at[j]], bufs[b],
                                      gsem[b]).wait()

            def scatter(j, b):
                pltpu.async_copy(bufs[b], acc.at[dst_v.at[j]], ssem[b],
                                 add=True)

            def scatter_wait(j, b):
                pltpu.make_async_copy(bufs[b], acc.at[dst_v.at[j]],
                                      ssem[b]).wait()

            gather(0, 0)
            gather(1, 1)
            gather_wait(0, 0)
            scatter(0, 0)
            gather(2, 2)
            gather_wait(1, 1)
            scatter(1, 1)
            gather(3, 3)

            def body(i, carry):
                j0 = 2 + i * 4
                for t in range(4):
                    j = j0 + t
                    b = (2 + t) % 4
                    gather_wait(j, b)
                    scatter(j, b)
                    scatter_wait(j - 2, (b + 2) % 4)
                    gather(j + 2, (b + 2) % 4)
                return carry

            lax.fori_loop(0, (CHUNKS - 2) // 4, body, 0)
            scatter_wait(CHUNKS - 2, (CHUNKS - 2) % 4)
            scatter_wait(CHUNKS - 1, (CHUNKS - 1) % 4)
            gather_wait(CHUNKS, CHUNKS % 4)
            gather_wait(CHUNKS + 1, (CHUNKS + 1) % 4)

        for p in range(npass):
            zero_acc()
            plsc.subcore_barrier()
            ring(ys[p])
            plsc.subcore_barrier()
            pltpu.sync_copy(acc.at[pl.ds(s * RPT, RPT)],
                            outs[p].at[c, pl.ds(s * RPT, RPT)])
            if p + 1 < npass:
                plsc.subcore_barrier()

    return _sc_agg


_sc_agg_l1 = _make_sc_agg(DOUT, 2)
_sc_agg_l2 = _make_sc_agg(DOUT, 1)


_BR = 1024


_DHH = DH // 2


def _tc_a_body(x_ref, w_ref, d_ref, oa_ref, ob_ref):
    deg = d_ref[:, 0:1] + d_ref[:, 1:2] + 1.0
    dinv = lax.rsqrt(deg)
    xw = jnp.dot(x_ref[...], w_ref[...], preferred_element_type=jnp.float32)
    y = xw * dinv
    oa_ref[...] = y[:, :_DHH]
    ob_ref[...] = y[:, _DHH:]


def _tc_a(x_pad, w1, degt):
    return pl.pallas_call(
        _tc_a_body,
        grid=(NP // _BR,),
        in_specs=[
            pl.BlockSpec((_BR, DIN), lambda i: (i, 0)),
            pl.BlockSpec((DIN, DH), lambda i: (0, 0)),
            pl.BlockSpec((_BR, NC), lambda i: (i, 0)),
        ],
        out_specs=[
            pl.BlockSpec((_BR, _DHH), lambda i: (i, 0)),
            pl.BlockSpec((_BR, _DHH), lambda i: (i, 0)),
        ],
        out_shape=[
            jax.ShapeDtypeStruct((NP, _DHH), jnp.float32),
            jax.ShapeDtypeStruct((NP, _DHH), jnp.float32),
        ],
    )(x_pad, w1, degt)


def _tc_b_body(pa_ref, pb_ref, ya_ref, yb_ref, d_ref, b_ref, w_ref, o_ref):
    deg = d_ref[:, 0:1] + d_ref[:, 1:2] + 1.0
    dinv = lax.rsqrt(deg)
    agga = pa_ref[0] + pa_ref[1] + ya_ref[...]
    aggb = pb_ref[0] + pb_ref[1] + yb_ref[...]
    ha = jnp.maximum(agga * dinv + b_ref[:, :_DHH], 0.0)
    hb = jnp.maximum(aggb * dinv + b_ref[:, _DHH:], 0.0)
    hw = (jnp.dot(ha, w_ref[:_DHH, :], preferred_element_type=jnp.float32)
          + jnp.dot(hb, w_ref[_DHH:, :], preferred_element_type=jnp.float32))
    o_ref[...] = hw * dinv


def _tc_b(p1a, p1b, y1a, y1b, degt, b1, w2):
    return pl.pallas_call(
        _tc_b_body,
        grid=(NP // _BR,),
        in_specs=[
            pl.BlockSpec((NC, _BR, _DHH), lambda i: (0, i, 0)),
            pl.BlockSpec((NC, _BR, _DHH), lambda i: (0, i, 0)),
            pl.BlockSpec((_BR, _DHH), lambda i: (i, 0)),
            pl.BlockSpec((_BR, _DHH), lambda i: (i, 0)),
            pl.BlockSpec((_BR, NC), lambda i: (i, 0)),
            pl.BlockSpec((1, DH), lambda i: (0, 0)),
            pl.BlockSpec((DH, DOUT), lambda i: (0, 0)),
        ],
        out_specs=pl.BlockSpec((_BR, DOUT), lambda i: (i, 0)),
        out_shape=jax.ShapeDtypeStruct((NP, DOUT), jnp.float32),
    )(p1a, p1b, y1a, y1b, degt, b1, w2)


_BRC = 1000


def _tc_c_body(p_ref, y_ref, d_ref, b_ref, o_ref):
    deg = d_ref[:, 0:1] + d_ref[:, 1:2] + 1.0
    dinv = lax.rsqrt(deg)
    agg = p_ref[0] + p_ref[1] + y_ref[...]
    o_ref[...] = agg * dinv + b_ref[...]


def _tc_c(p2, y2, degt, b2):
    return pl.pallas_call(
        _tc_c_body,
        grid=(N // _BRC,),
        in_specs=[
            pl.BlockSpec((NC, _BRC, DOUT), lambda i: (0, i, 0)),
            pl.BlockSpec((_BRC, DOUT), lambda i: (i, 0)),
            pl.BlockSpec((_BRC, NC), lambda i: (i, 0)),
            pl.BlockSpec((1, DOUT), lambda i: (0, 0)),
        ],
        out_specs=pl.BlockSpec((_BRC, DOUT), lambda i: (i, 0)),
        out_shape=jax.ShapeDtypeStruct((N, DOUT), jnp.float32),
    )(p2, y2, degt, b2)


def kernel(x, edge_index, W1, b1, W2, b2):
    src = edge_index[0]
    dst = edge_index[1]
    fill = jnp.full((EPT - E,), N, dtype=jnp.int32)
    src_p = jnp.concatenate([src, fill]).reshape(NW, CHUNKS, C)
    dst_p = jnp.concatenate([dst, fill]).reshape(NW, CHUNKS, C)
    x_pad = jnp.pad(x, ((0, NP - N), (0, 0)))

    degp = _sc_deg(dst_p)
    degt = degp.T
    y1a, y1b = _tc_a(x_pad, W1, degt)
    p1a, p1b = _sc_agg_l1(y1a, y1b, src_p, dst_p)
    y2 = _tc_b(p1a, p1b, y1a, y1b, degt, b1.reshape(1, DH), W2)
    (p2,) = _sc_agg_l2(y2, src_p, dst_p)
    out = _tc_c(p2, y2, degt, b2.reshape(1, DOUT))
    return out


# speedup vs baseline: 1.1261x; 1.0012x over previous
import functools

import jax
import jax.numpy as jnp
from jax import lax
from jax.experimental import pallas as pl
from jax.experimental.pallas import tpu as pltpu, tpu_sc as plsc

N = 10000
E = 640000
DIN, DH, DOUT = 128, 128, 64

NC, NS = 2, 16
NW = NC * NS
C = 128
CHUNKS = 158
EW = CHUNKS * C
EPT = NW * EW
NP = 10240
RPT = NP // NS

_mesh = plsc.VectorSubcoreMesh(core_axis_name="c", subcore_axis_name="s")


@functools.partial(
    pl.kernel,
    out_type=jax.ShapeDtypeStruct((NC, NP), jnp.float32),
    mesh=_mesh,
    scratch_types=[
        pltpu.VMEM((CHUNKS, C), jnp.int32),
        pltpu.VMEM((C,), jnp.float32),
        pltpu.VMEM((C,), jnp.float32),
        pltpu.VMEM_SHARED((NP,), jnp.float32),
    ],
)
def _sc_deg(dst_hbm, out_hbm, idx_v, ones_v, zero_v, acc):
    c = lax.axis_index("c")
    s = lax.axis_index("s")
    wid = s * NC + c
    pltpu.sync_copy(dst_hbm.at[wid], idx_v)
    for k in range(C // 16):
        ones_v[pl.ds(k * 16, 16)] = jnp.ones((16,), jnp.float32)
        zero_v[pl.ds(k * 16, 16)] = jnp.zeros((16,), jnp.float32)
    for k in range(RPT // C):
        pltpu.sync_copy(zero_v, acc.at[pl.ds(s * RPT + k * C, C)])
    plsc.subcore_barrier()

    def body(j, carry):
        pltpu.sync_copy(ones_v, acc.at[idx_v.at[j]], add=True)
        return carry

    lax.fori_loop(0, CHUNKS, body, 0)
    plsc.subcore_barrier()
    pltpu.sync_copy(acc.at[pl.ds(s * RPT, RPT)],
                    out_hbm.at[c, pl.ds(s * RPT, RPT)])


def _make_sc_agg(d, npass):
    @functools.partial(
        pl.kernel,
        out_type=[jax.ShapeDtypeStruct((NC, NP, d), jnp.float32)] * npass,
        mesh=_mesh,
        scratch_types=[
            pltpu.VMEM((CHUNKS + 2, C), jnp.int32),
            pltpu.VMEM((CHUNKS, C), jnp.int32),
            [pltpu.VMEM((C, d), jnp.float32)] * 4,
            pltpu.VMEM((C, d), jnp.float32),
            [pltpu.SemaphoreType.DMA] * 4,
            [pltpu.SemaphoreType.DMA] * 4,
            pltpu.VMEM_SHARED((NP, d), jnp.float32),
        ],
        compiler_params=pltpu.CompilerParams(use_tc_tiling_on_sc=False),
    )
    def _sc_agg(*refs):
        ys = refs[:npass]
        src_hbm, dst_hbm = refs[npass], refs[npass + 1]
        outs = refs[npass + 2:2 * npass + 2]
        src_v, dst_v, bufs, zbuf, gsem, ssem, acc = refs[2 * npass + 2:]
        c = lax.axis_index("c")
        s = lax.axis_index("s")
        wid = s * NC + c
        pltpu.sync_copy(src_hbm.at[wid], src_v.at[pl.ds(0, CHUNKS)])
        pltpu.sync_copy(dst_hbm.at[wid], dst_v)
        for k in range(C // 16):
            src_v[CHUNKS, pl.ds(k * 16, 16)] = jnp.zeros((16,), jnp.int32)
            src_v[CHUNKS + 1, pl.ds(k * 16, 16)] = jnp.zeros((16,), jnp.int32)

        def zrow(r, carry):
            for k in range(d // 16):
                zbuf[r, pl.ds(k * 16, 16)] = jnp.zeros((16,), jnp.float32)
            return carry

        lax.fori_loop(0, C, zrow, 0)

        def zero_acc():
            for k in range(RPT // C):
                pltpu.sync_copy(zbuf, acc.at[pl.ds(s * RPT + k * C, C)])

        def make_ops(y_hbm):
            def gather(j, b):
                pltpu.async_copy(y_hbm.at[src_v.at[j]], bufs[b], gsem[b])

            def gather_wait(j, b):
                pltpu.make_async_copy(y_hbm.at[src_v.at[j]], bufs[b],
                                      gsem[b]).wait()

            def scatter(j, b):
                pltpu.async_copy(bufs[b], acc.at[dst_v.at[j]], ssem[b],
                                 add=True)

            def scatter_wait(j, b):
                pltpu.make_async_copy(bufs[b], acc.at[dst_v.at[j]],
                                      ssem[b]).wait()

            return gather, gather_wait, scatter, scatter_wait

        def prime(y_hbm):
            gather, _, _, _ = make_ops(y_hbm)
            gather(0, 0)
            gather(1, 1)

        def ring(y_hbm):
            gather, gather_wait, scatter, scatter_wait = make_ops(y_hbm)
            gather_wait(0, 0)
            scatter(0, 0)
            gather(2, 2)
            gather_wait(1, 1)
            scatter(1, 1)
            gather(3, 3)

            def body(i, carry):
                j0 = 2 + i * 4
                for t in range(4):
                    j = j0 + t
                    b = (2 + t) % 4
                    gather_wait(j, b)
                    scatter(j, b)
                    scatter_wait(j - 2, (b + 2) % 4)
                    gather(j + 2, (b + 2) % 4)
                return carry

            lax.fori_loop(0, (CHUNKS - 2) // 4, body, 0)
            scatter_wait(CHUNKS - 2, (CHUNKS - 2) % 4)
            scatter_wait(CHUNKS - 1, (CHUNKS - 1) % 4)
            gather_wait(CHUNKS, CHUNKS % 4)
            gather_wait(CHUNKS + 1, (CHUNKS + 1) % 4)

        for p in range(npass):
            prime(ys[p])
            if p > 0:
                plsc.subcore_barrier()
                pltpu.sync_copy(acc.at[pl.ds(s * RPT, RPT)],
                                outs[p - 1].at[c, pl.ds(s * RPT, RPT)])
            zero_acc()
            plsc.subcore_barrier()
            ring(ys[p])
        plsc.subcore_barrier()
        pltpu.sync_copy(acc.at[pl.ds(s * RPT, RPT)],
                        outs[npass - 1].at[c, pl.ds(s * RPT, RPT)])

    return _sc_agg


_sc_agg_l1 = _make_sc_agg(DOUT, 2)
_sc_agg_l2 = _make_sc_agg(DOUT, 1)


_BR = 1024


_DHH = DH // 2


def _tc_a_body(x_ref, w_ref, d_ref, oa_ref, ob_ref):
    deg = d_ref[:, 0:1] + d_ref[:, 1:2] + 1.0
    dinv = lax.rsqrt(deg)
    xw = jnp.dot(x_ref[...], w_ref[...], preferred_element_type=jnp.float32)
    y = xw * dinv
    oa_ref[...] = y[:, :_DHH]
    ob_ref[...] = y[:, _DHH:]


def _tc_a(x_pad, w1, degt):
    return pl.pallas_call(
        _tc_a_body,
        grid=(NP // _BR,),
        in_specs=[
            pl.BlockSpec((_BR, DIN), lambda i: (i, 0)),
            pl.BlockSpec((DIN, DH), lambda i: (0, 0)),
            pl.BlockSpec((_BR, NC), lambda i: (i, 0)),
        ],
        out_specs=[
            pl.BlockSpec((_BR, _DHH), lambda i: (i, 0)),
            pl.BlockSpec((_BR, _DHH), lambda i: (i, 0)),
        ],
        out_shape=[
            jax.ShapeDtypeStruct((NP, _DHH), jnp.float32),
            jax.ShapeDtypeStruct((NP, _DHH), jnp.float32),
        ],
    )(x_pad, w1, degt)


def _tc_b_body(pa_ref, pb_ref, ya_ref, yb_ref, d_ref, b_ref, w_ref, o_ref):
    deg = d_ref[:, 0:1] + d_ref[:, 1:2] + 1.0
    dinv = lax.rsqrt(deg)
    agga = pa_ref[0] + pa_ref[1] + ya_ref[...]
    aggb = pb_ref[0] + pb_ref[1] + yb_ref[...]
    ha = jnp.maximum(agga * dinv + b_ref[:, :_DHH], 0.0)
    hb = jnp.maximum(aggb * dinv + b_ref[:, _DHH:], 0.0)
    hw = (jnp.dot(ha, w_ref[:_DHH, :], preferred_element_type=jnp.float32)
          + jnp.dot(hb, w_ref[_DHH:, :], preferred_element_type=jnp.float32))
    o_ref[...] = hw * dinv


def _tc_b(p1a, p1b, y1a, y1b, degt, b1, w2):
    return pl.pallas_call(
        _tc_b_body,
        grid=(NP // _BR,),
        in_specs=[
            pl.BlockSpec((NC, _BR, _DHH), lambda i: (0, i, 0)),
            pl.BlockSpec((NC, _BR, _DHH), lambda i: (0, i, 0)),
            pl.BlockSpec((_BR, _DHH), lambda i: (i, 0)),
            pl.BlockSpec((_BR, _DHH), lambda i: (i, 0)),
            pl.BlockSpec((_BR, NC), lambda i: (i, 0)),
            pl.BlockSpec((1, DH), lambda i: (0, 0)),
            pl.BlockSpec((DH, DOUT), lambda i: (0, 0)),
        ],
        out_specs=pl.BlockSpec((_BR, DOUT), lambda i: (i, 0)),
        out_shape=jax.ShapeDtypeStruct((NP, DOUT), jnp.float32),
    )(p1a, p1b, y1a, y1b, degt, b1, w2)


_BRC = 1000


def _tc_c_body(p_ref, y_ref, d_ref, b_ref, o_ref):
    deg = d_ref[:, 0:1] + d_ref[:, 1:2] + 1.0
    dinv = lax.rsqrt(deg)
    agg = p_ref[0] + p_ref[1] + y_ref[...]
    o_ref[...] = agg * dinv + b_ref[...]


def _tc_c(p2, y2, degt, b2):
    return pl.pallas_call(
        _tc_c_body,
        grid=(N // _BRC,),
        in_specs=[
            pl.BlockSpec((NC, _BRC, DOUT), lambda i: (0, i, 0)),
            pl.BlockSpec((_BRC, DOUT), lambda i: (i, 0)),
            pl.BlockSpec((_BRC, NC), lambda i: (i, 0)),
            pl.BlockSpec((1, DOUT), lambda i: (0, 0)),
        ],
        out_specs=pl.BlockSpec((_BRC, DOUT), lambda i: (i, 0)),
        out_shape=jax.ShapeDtypeStruct((N, DOUT), jnp.float32),
    )(p2, y2, degt, b2)


def kernel(x, edge_index, W1, b1, W2, b2):
    src = edge_index[0]
    dst = edge_index[1]
    fill = jnp.full((EPT - E,), N, dtype=jnp.int32)
    src_p = jnp.concatenate([src, fill]).reshape(NW, CHUNKS, C)
    dst_p = jnp.concatenate([dst, fill]).reshape(NW, CHUNKS, C)
    x_pad = jnp.pad(x, ((0, NP - N), (0, 0)))

    degp = _sc_deg(dst_p)
    degt = degp.T
    y1a, y1b = _tc_a(x_pad, W1, degt)
    p1a, p1b = _sc_agg_l1(y1a, y1b, src_p, dst_p)
    y2 = _tc_b(p1a, p1b, y1a, y1b, degt, b1.reshape(1, DH), W2)
    (p2,) = _sc_agg_l2(y2, src_p, dst_p)
    out = _tc_c(p2, y2, degt, b2.reshape(1, DOUT))
    return out


# speedup vs baseline: 1.1495x; 1.0208x over previous
import functools

import jax
import jax.numpy as jnp
from jax import lax
from jax.experimental import pallas as pl
from jax.experimental.pallas import tpu as pltpu, tpu_sc as plsc

N = 10000
E = 640000
DIN, DH, DOUT = 128, 128, 64

NC, NS = 2, 16
NW = NC * NS
C = 128
CHUNKS = 158
EW = CHUNKS * C
EPT = NW * EW
NP = 10240
RPT = NP // NS

_mesh = plsc.VectorSubcoreMesh(core_axis_name="c", subcore_axis_name="s")


@functools.partial(
    pl.kernel,
    out_type=jax.ShapeDtypeStruct((NC, NP), jnp.float32),
    mesh=_mesh,
    scratch_types=[
        pltpu.VMEM((CHUNKS, C), jnp.int32),
        pltpu.VMEM((C,), jnp.float32),
        pltpu.VMEM((C,), jnp.float32),
        pltpu.VMEM_SHARED((NP,), jnp.float32),
    ],
)
def _sc_deg(dst_hbm, out_hbm, idx_v, ones_v, zero_v, acc):
    c = lax.axis_index("c")
    s = lax.axis_index("s")
    wid = s * NC + c
    pltpu.sync_copy(dst_hbm.at[wid], idx_v)
    for k in range(C // 16):
        ones_v[pl.ds(k * 16, 16)] = jnp.ones((16,), jnp.float32)
        zero_v[pl.ds(k * 16, 16)] = jnp.zeros((16,), jnp.float32)
    for k in range(RPT // C):
        pltpu.sync_copy(zero_v, acc.at[pl.ds(s * RPT + k * C, C)])
    plsc.subcore_barrier()

    def body(j, carry):
        pltpu.sync_copy(ones_v, acc.at[idx_v.at[j]], add=True)
        return carry

    lax.fori_loop(0, CHUNKS, body, 0)
    plsc.subcore_barrier()
    pltpu.sync_copy(acc.at[pl.ds(s * RPT, RPT)],
                    out_hbm.at[c, pl.ds(s * RPT, RPT)])


def _make_sc_agg(d, npass):
    @functools.partial(
        pl.kernel,
        out_type=[jax.ShapeDtypeStruct((NC, NP, d), jnp.float32)] * npass,
        mesh=_mesh,
        scratch_types=[
            pltpu.VMEM((CHUNKS + 2, C), jnp.int32),
            pltpu.VMEM((CHUNKS, C), jnp.int32),
            [pltpu.VMEM((C, d), jnp.float32)] * 4,
            pltpu.VMEM((C, d), jnp.float32),
            [pltpu.SemaphoreType.DMA] * 4,
            [pltpu.SemaphoreType.DMA] * 4,
            pltpu.VMEM_SHARED((NP, d), jnp.float32),
        ],
        compiler_params=pltpu.CompilerParams(use_tc_tiling_on_sc=False),
    )
    def _sc_agg(*refs):
        ys = refs[:npass]
        src_hbm, dst_hbm = refs[npass], refs[npass + 1]
        outs = refs[npass + 2:2 * npass + 2]
        src_v, dst_v, bufs, zbuf, gsem, ssem, acc = refs[2 * npass + 2:]
        c = lax.axis_index("c")
        s = lax.axis_index("s")
        wid = s * NC + c
        pltpu.sync_copy(src_hbm.at[wid], src_v.at[pl.ds(0, CHUNKS)])
        pltpu.sync_copy(dst_hbm.at[wid], dst_v)
        for k in range(C // 16):
            src_v[CHUNKS, pl.ds(k * 16, 16)] = jnp.zeros((16,), jnp.int32)
            src_v[CHUNKS + 1, pl.ds(k * 16, 16)] = jnp.zeros((16,), jnp.int32)

        def zrow(r, carry):
            for k in range(d // 16):
                zbuf[r, pl.ds(k * 16, 16)] = jnp.zeros((16,), jnp.float32)
            return carry

        lax.fori_loop(0, C, zrow, 0)

        def zero_acc():
            for k in range(RPT // C):
                pltpu.sync_copy(zbuf, acc.at[pl.ds(s * RPT + k * C, C)])

        def make_ops(y_hbm):
            def gather(j, b):
                pltpu.async_copy(y_hbm.at[src_v.at[j]], bufs[b], gsem[b])

            def gather_wait(j, b):
                pltpu.make_async_copy(y_hbm.at[src_v.at[j]], bufs[b],
                                      gsem[b]).wait()

            def scatter(j, b):
                pltpu.async_copy(bufs[b], acc.at[dst_v.at[j]], ssem[b],
                                 add=True)

            def scatter_wait(j, b):
                pltpu.make_async_copy(bufs[b], acc.at[dst_v.at[j]],
                                      ssem[b]).wait()

            return gather, gather_wait, scatter, scatter_wait

        def prime(y_hbm):
            gather, _, _, _ = make_ops(y_hbm)
            gather(0, 0)
            gather(1, 1)

        def ring(y_hbm):
            gather, gather_wait, scatter, scatter_wait = make_ops(y_hbm)
            gather_wait(0, 0)
            scatter(0, 0)
            gather(2, 2)
            gather_wait(1, 1)
            scatter(1, 1)
            gather(3, 3)

            def body(i, carry):
                j0 = 2 + i * 4
                for t in range(4):
                    j = j0 + t
                    b = (2 + t) % 4
                    gather_wait(j, b)
                    scatter(j, b)
                    scatter_wait(j - 2, (b + 2) % 4)
                    gather(j + 2, (b + 2) % 4)
                return carry

            lax.fori_loop(0, (CHUNKS - 2) // 4, body, 0)
            scatter_wait(CHUNKS - 2, (CHUNKS - 2) % 4)
            scatter_wait(CHUNKS - 1, (CHUNKS - 1) % 4)
            gather_wait(CHUNKS, CHUNKS % 4)
            gather_wait(CHUNKS + 1, (CHUNKS + 1) % 4)

        for p in range(npass):
            prime(ys[p])
            if p > 0:
                plsc.subcore_barrier()
                pltpu.sync_copy(acc.at[pl.ds(s * RPT, RPT)],
                                outs[p - 1].at[c, pl.ds(s * RPT, RPT)])
            zero_acc()
            plsc.subcore_barrier()
            ring(ys[p])
        plsc.subcore_barrier()
        pltpu.sync_copy(acc.at[pl.ds(s * RPT, RPT)],
                        outs[npass - 1].at[c, pl.ds(s * RPT, RPT)])

    return _sc_agg


_sc_agg_l1 = _make_sc_agg(DOUT, 2)
_sc_agg_l2 = _make_sc_agg(DOUT, 1)


_BR = 1024


_DHH = DH // 2


def _tc_a_body(x_ref, w_ref, d_ref, oa_ref, ob_ref):
    deg = d_ref[:, 0:1] + d_ref[:, 1:2] + 1.0
    dinv = lax.rsqrt(deg)
    xw = jnp.dot(x_ref[...], w_ref[...], preferred_element_type=jnp.float32)
    y = xw * dinv
    oa_ref[...] = y[:, :_DHH]
    ob_ref[...] = y[:, _DHH:]


def _tc_a(x_pad, w1, degt):
    return pl.pallas_call(
        _tc_a_body,
        grid=(NP // _BR,),
        in_specs=[
            pl.BlockSpec((_BR, DIN), lambda i: (i, 0)),
            pl.BlockSpec((DIN, DH), lambda i: (0, 0)),
            pl.BlockSpec((_BR, NC), lambda i: (i, 0)),
        ],
        out_specs=[
            pl.BlockSpec((_BR, _DHH), lambda i: (i, 0)),
            pl.BlockSpec((_BR, _DHH), lambda i: (i, 0)),
        ],
        out_shape=[
            jax.ShapeDtypeStruct((NP, _DHH), jnp.float32),
            jax.ShapeDtypeStruct((NP, _DHH), jnp.float32),
        ],
    )(x_pad, w1, degt)


def _tc_b_body(pa_ref, pb_ref, ya_ref, yb_ref, d_ref, b_ref, w_ref, o_ref):
    deg = d_ref[:, 0:1] + d_ref[:, 1:2] + 1.0
    dinv = lax.rsqrt(deg)
    agga = pa_ref[0] + pa_ref[1] + ya_ref[...]
    aggb = pb_ref[0] + pb_ref[1] + yb_ref[...]
    ha = jnp.maximum(agga * dinv + b_ref[:, :_DHH], 0.0)
    hb = jnp.maximum(aggb * dinv + b_ref[:, _DHH:], 0.0)
    hw = (jnp.dot(ha, w_ref[:_DHH, :], preferred_element_type=jnp.float32)
          + jnp.dot(hb, w_ref[_DHH:, :], preferred_element_type=jnp.float32))
    o_ref[...] = hw * dinv


def _tc_b(p1a, p1b, y1a, y1b, degt, b1, w2):
    return pl.pallas_call(
        _tc_b_body,
        grid=(NP // _BR,),
        in_specs=[
            pl.BlockSpec((NC, _BR, _DHH), lambda i: (0, i, 0)),
            pl.BlockSpec((NC, _BR, _DHH), lambda i: (0, i, 0)),
            pl.BlockSpec((_BR, _DHH), lambda i: (i, 0)),
            pl.BlockSpec((_BR, _DHH), lambda i: (i, 0)),
            pl.BlockSpec((_BR, NC), lambda i: (i, 0)),
            pl.BlockSpec((1, DH), lambda i: (0, 0)),
            pl.BlockSpec((DH, DOUT), lambda i: (0, 0)),
        ],
        out_specs=pl.BlockSpec((_BR, DOUT), lambda i: (i, 0)),
        out_shape=jax.ShapeDtypeStruct((NP, DOUT), jnp.float32),
    )(p1a, p1b, y1a, y1b, degt, b1, w2)


_BRC = 1000


def _tc_c_body(p_ref, y_ref, d_ref, b_ref, o_ref):
    deg = d_ref[:, 0:1] + d_ref[:, 1:2] + 1.0
    dinv = lax.rsqrt(deg)
    agg = p_ref[0] + p_ref[1] + y_ref[...]
    o_ref[...] = agg * dinv + b_ref[...]


def _tc_c(p2, y2, degt, b2):
    return pl.pallas_call(
        _tc_c_body,
        grid=(N // _BRC,),
        in_specs=[
            pl.BlockSpec((NC, _BRC, DOUT), lambda i: (0, i, 0)),
            pl.BlockSpec((_BRC, DOUT), lambda i: (i, 0)),
            pl.BlockSpec((_BRC, NC), lambda i: (i, 0)),
            pl.BlockSpec((1, DOUT), lambda i: (0, 0)),
        ],
        out_specs=pl.BlockSpec((_BRC, DOUT), lambda i: (i, 0)),
        out_shape=jax.ShapeDtypeStruct((N, DOUT), jnp.float32),
    )(p2, y2, degt, b2)


def kernel(x, edge_index, W1, b1, W2, b2):
    src = edge_index[0]
    dst = edge_index[1]
    fill = jnp.full((EPT - E,), N, dtype=jnp.int32)
    src_p = jnp.concatenate([src, fill]).reshape(NW, CHUNKS, C)
    dst_p = jnp.concatenate([dst, fill]).reshape(NW, CHUNKS, C)
    degp = _sc_deg(dst_p)
    degt = degp.T
    y1a, y1b = _tc_a(x, W1, degt)
    p1a, p1b = _sc_agg_l1(y1a, y1b, src_p, dst_p)
    y2 = _tc_b(p1a, p1b, y1a, y1b, degt, b1.reshape(1, DH), W2)
    (p2,) = _sc_agg_l2(y2, src_p, dst_p)
    out = _tc_c(p2, y2, degt, b2.reshape(1, DOUT))
    return out
